# baseline (speedup 1.0000x reference)
import functools

import jax
import jax.numpy as jnp
from jax import lax
from jax.experimental import pallas as pl
from jax.experimental.pallas import tpu as pltpu
from jax.experimental.pallas import tpu_sc as plsc

NUM_EMB = 100000
DIM = 300
B_TOTAL = 4096 * 200

_info = plsc.get_sparse_core_info()
_NC, _NS = _info.num_cores, _info.num_subcores
_NW = _NC * _NS

_DIM_PAD = 304
_CHUNK = 128
_PER_W = B_TOTAL // _NW
_NSTEP = _PER_W // _CHUNK


def _sc_gather(x_flat, table_pad):
    mesh = plsc.VectorSubcoreMesh(core_axis_name="c", subcore_axis_name="s")

    @functools.partial(
        pl.kernel,
        out_type=jax.ShapeDtypeStruct((B_TOTAL * DIM,), jnp.float32),
        mesh=mesh,
        scratch_types=[
            pltpu.VMEM((_CHUNK,), jnp.int32),
            pltpu.VMEM((_CHUNK, _DIM_PAD), jnp.float32),
            pltpu.VMEM((_CHUNK * DIM + 16,), jnp.float32),
            pltpu.SemaphoreType.DMA,
        ],
        compiler_params=pltpu.CompilerParams(use_tc_tiling_on_sc=False),
    )
    def k(x_hbm, table_hbm, out_hbm, idx_v, rows_v, packed_v, sem):
        wid = lax.axis_index("s") * _NC + lax.axis_index("c")
        wbase = wid * _PER_W

        def body(i, carry):
            base = wbase + i * _CHUNK
            pltpu.sync_copy(x_hbm.at[pl.ds(base, _CHUNK)], idx_v)
            pltpu.async_copy(table_hbm.at[idx_v], rows_v, sem).wait()

            def pack_row(c, carry2):
                dst = c * DIM
                for j in range(DIM // 16 + 1):
                    v = rows_v[c, pl.ds(j * 16, 16)]
                    packed_v[pl.ds(dst + j * 16, 16)] = v
                return carry2

            lax.fori_loop(0, _CHUNK, pack_row, 0)
            pltpu.sync_copy(packed_v.at[pl.ds(0, _CHUNK * DIM)],
                            out_hbm.at[pl.ds(base * DIM, _CHUNK * DIM)])
            return carry

        lax.fori_loop(0, _NSTEP, body, 0)

    return k(x_flat, table_pad)


def kernel(x, table):
    x_flat = x.reshape(-1).astype(jnp.int32)
    table_pad = jnp.pad(table, ((0, 0), (0, _DIM_PAD - DIM)))
    out = _sc_gather(x_flat, table_pad)
    return out.reshape(x.shape[0], x.shape[1], DIM)


# speedup vs baseline: 2.4980x; 2.4980x over previous
import functools

import jax
import jax.numpy as jnp
from jax import lax
from jax.experimental import pallas as pl
from jax.experimental.pallas import tpu as pltpu
from jax.experimental.pallas import tpu_sc as plsc

NUM_EMB = 100000
DIM = 300
DIM_PAD = 384
B_TOTAL = 4096 * 200

_info = plsc.get_sparse_core_info()
_NC, _NS = _info.num_cores, _info.num_subcores
_NW = _NC * _NS

_CHUNK = 80
_PER_W = B_TOTAL // _NW
_NSTEP = _PER_W // _CHUNK


def _sc_gather(x_flat, table_pad):
    mesh = plsc.VectorSubcoreMesh(core_axis_name="c", subcore_axis_name="s")

    @functools.partial(
        pl.kernel,
        out_type=jax.ShapeDtypeStruct((B_TOTAL, DIM), jnp.float32),
        mesh=mesh,
        scratch_types=[
            pltpu.VMEM((_CHUNK,), jnp.int32),
            pltpu.VMEM((_CHUNK,), jnp.int32),
            pltpu.VMEM((_CHUNK, DIM_PAD), jnp.float32),
            pltpu.VMEM((_CHUNK, DIM_PAD), jnp.float32),
            pltpu.VMEM((_CHUNK, DIM), jnp.float32),
            pltpu.VMEM((_CHUNK, DIM), jnp.float32),
            pltpu.SemaphoreType.DMA,
            pltpu.SemaphoreType.DMA,
            pltpu.SemaphoreType.DMA,
            pltpu.SemaphoreType.DMA,
            pltpu.SemaphoreType.DMA,
            pltpu.SemaphoreType.DMA,
        ],
    )
    def k(x_hbm, table_hbm, out_hbm,
          ibuf0, ibuf1, grow0, grow1, rows0, rows1,
          isem0, isem1, gsem0, gsem1, wsem0, wsem1):
        wid = lax.axis_index("s") * _NC + lax.axis_index("c")
        wbase = wid * _PER_W

        ibuf = (ibuf0, ibuf1)
        grow = (grow0, grow1)
        rows = (rows0, rows1)
        isem = (isem0, isem1)
        gsem = (gsem0, gsem1)
        wsem = (wsem0, wsem1)

        def issue_idx(step, p):
            pltpu.async_copy(
                x_hbm.at[pl.ds(wbase + step * _CHUNK, _CHUNK)],
                ibuf[p], isem[p])

        def wait_idx(p):
            pltpu.make_async_copy(
                x_hbm.at[pl.ds(wbase, _CHUNK)], ibuf[p], isem[p]).wait()

        def issue_gather(p):
            pltpu.async_copy(table_hbm.at[ibuf[p]], grow[p], gsem[p])

        def wait_gather(p):
            pltpu.make_async_copy(
                table_hbm.at[ibuf[p]], grow[p], gsem[p]).wait()

        def rearrange_vec(p):
            offs = tuple(range(0, 256, 16)) + (256, 272, 284)

            def body(c, carry):
                for off in offs:
                    rows[p][c, pl.ds(off, 16)] = grow[p][c, pl.ds(off, 16)]
                return carry

            lax.fori_loop(0, _CHUNK, body, 0)

        def issue_wb(step, p):
            pltpu.async_copy(
                rows[p],
                out_hbm.at[pl.ds(wbase + step * _CHUNK, _CHUNK)], wsem[p])

        def wait_wb(p):
            pltpu.make_async_copy(
                rows[p], out_hbm.at[pl.ds(wbase, _CHUNK)], wsem[p]).wait()

        def step_body(i, p, first, last):
            wait_gather(p)
            if not last:
                wait_idx(1 - p)
                issue_gather(1 - p)
            if i + 2 < _NSTEP:
                issue_idx(i + 2, p)
            if not first:
                wait_wb(p)
            rearrange_vec(p)
            issue_wb(i, p)

        issue_idx(0, 0)
        issue_idx(1, 1)
        wait_idx(0)
        issue_gather(0)
        step_body(0, 0, True, False)
        step_body(1, 1, True, False)

        def steady(g, carry):
            i = 2 + 2 * g

            def one(i, p):
                wait_gather(p)
                wait_idx(1 - p)
                issue_gather(1 - p)
                pl.when(i + 2 < _NSTEP)(lambda: issue_idx(i + 2, p))
                wait_wb(p)
                rearrange_vec(p)
                issue_wb(i, p)

            one(i, 0)
            one(i + 1, 1)
            return carry

        lax.fori_loop(0, (_NSTEP - 4) // 2, steady, 0)

        step_body(_NSTEP - 2, 0, False, False)
        step_body(_NSTEP - 1, 1, False, True)

        wait_wb(0)
        wait_wb(1)

    return k(x_flat, table_pad)


def kernel(x, table):
    x_flat = x.reshape(-1).astype(jnp.int32)
    table_pad = jnp.pad(table, ((0, 0), (0, DIM_PAD - DIM)))
    out = _sc_gather(x_flat, table_pad)
    return out.reshape(x.shape[0], x.shape[1], DIM)
